# baseline (speedup 1.0000x reference)
import functools

import numpy as np
import jax
import jax.numpy as jnp
from jax import lax
from jax.experimental import pallas as pl
from jax.experimental.pallas import tpu as pltpu
from jax.experimental.pallas import tpu_sc as plsc

N = 1024
M = 512
DV = 3
DC = 6
E = N * DV
NUM_ITER = 5
BITS_PER_SYM = 2
CODERATE = 0.5

NC = 2
NS = 16
NW = NC * NS
L = 16


def _build_tables():
    rng = np.random.RandomState(0)
    cn = rng.permutation(np.repeat(np.arange(M), DC))
    order = np.argsort(cn, kind="stable")
    perm = np.empty(E, np.int64)
    for m in range(M):
        for j in range(DC):
            perm[j * M + m] = order[m * DC + j]
    vn_of = perm // DV
    pos_of_orig = np.empty(E, np.int64)
    pos_of_orig[perm] = np.arange(E)
    pos3 = pos_of_orig.reshape(N, DV).T.copy()
    return (perm.astype(np.int32), vn_of.astype(np.int32), pos3.astype(np.int32))


_PERM_NP, _VNOF_NP, _POS3_NP = _build_tables()


def _phi(x):
    x = jnp.clip(x, 1e-7, 20.0)
    return -jnp.log(jnp.tanh(x * 0.5))


def _half_swap(y):
    return jnp.concatenate([y[:, 4:], y[:, :4]], axis=1)


def _lo_both(y):
    return jnp.concatenate([y[:, :4], y[:, :4]], axis=1)


def _hi_both(y):
    return jnp.concatenate([y[:, 4:], y[:, 4:]], axis=1)


def _cn_math(x, out_ref):
    t = jnp.where(x < 0, -1.0, 1.0)
    ph = _phi(jnp.abs(x))
    ph_s = ph[:, 0] + _half_swap(ph[:, 0])
    ph_s = ph_s + _lo_both(ph[:, 1])
    ph_s = ph_s + _hi_both(ph[:, 1])
    ph_s = ph_s + _lo_both(ph[:, 2])
    ph_s = ph_s + _hi_both(ph[:, 2])
    tp = t[:, 0] * t[:, 1] * t[:, 2]
    t_s = tp * _half_swap(tp)
    for d in range(3):
        out_ref[:, d] = (t_s * t[:, d]) * _phi(ph_s - ph[:, d])


def _cn_body(msg_ref, w_ref, out_ref):
    _cn_math(msg_ref[...] * w_ref[...], out_ref)


def _cn_loss_body(msg_ref, w_ref, lo_ref, out_ref, part_ref):
    _cn_math(msg_ref[...] * w_ref[...], out_ref)
    z = -lo_ref[...]
    sp = jnp.maximum(z, 0.0) + jnp.log(1.0 + jnp.exp(-jnp.abs(z)))
    part_ref[...] = jnp.reshape(jnp.sum(sp), (1, 1, 1))


def _cn_update(msg4, w4, bb=128):
    b = msg4.shape[0]
    return pl.pallas_call(
        _cn_body,
        grid=(b // bb,),
        in_specs=[pl.BlockSpec((bb, 3, 8, 128), lambda i: (i, 0, 0, 0)),
                  pl.BlockSpec((1, 3, 8, 128), lambda i: (0, 0, 0, 0))],
        out_specs=pl.BlockSpec((bb, 3, 8, 128), lambda i: (i, 0, 0, 0)),
        out_shape=jax.ShapeDtypeStruct((b, 3, 8, 128), jnp.float32),
    )(msg4, w4)


def _cn_update_loss(msg4, w4, lo3, bb=128):
    b = msg4.shape[0]
    return pl.pallas_call(
        _cn_loss_body,
        grid=(b // bb,),
        in_specs=[pl.BlockSpec((bb, 3, 8, 128), lambda i: (i, 0, 0, 0)),
                  pl.BlockSpec((1, 3, 8, 128), lambda i: (0, 0, 0, 0)),
                  pl.BlockSpec((bb, 8, 128), lambda i: (i, 0, 0))],
        out_specs=(pl.BlockSpec((bb, 3, 8, 128), lambda i: (i, 0, 0, 0)),
                   pl.BlockSpec((1, 1, 1), lambda i: (i, 0, 0))),
        out_shape=(jax.ShapeDtypeStruct((b, 3, 8, 128), jnp.float32),
                   jax.ShapeDtypeStruct((b // bb, 1, 1), jnp.float32)),
    )(msg4, w4, lo3)


def _loss_body(*refs):
    out_ref = refs[-1]
    s = jnp.float32(0.0)
    for r in refs[:-1]:
        x = -r[...]
        sp = jnp.maximum(x, 0.0) + jnp.log(1.0 + jnp.exp(-jnp.abs(x)))
        s = s + jnp.sum(sp)
    out_ref[...] = jnp.reshape(s, (1, 1, 1))


def _loss_partials(llr_outs, bb=256):
    b = llr_outs[0].shape[0]
    g = b // bb
    return pl.pallas_call(
        _loss_body,
        grid=(g,),
        in_specs=[pl.BlockSpec((bb, 8, 128), lambda i: (i, 0, 0))
                  for _ in llr_outs],
        out_specs=pl.BlockSpec((1, 1, 1), lambda i: (i, 0, 0)),
        out_shape=jax.ShapeDtypeStruct((g, 1, 1), jnp.float32),
    )(*llr_outs)


_GI = 8
_GS = 8


def _sc_init_build(batch):
    rows_w = batch // NW
    ngroups = rows_w // _GI
    mesh = plsc.VectorSubcoreMesh(core_axis_name="c", subcore_axis_name="s")

    @functools.partial(
        pl.kernel,
        out_type=(
            jax.ShapeDtypeStruct((batch * N,), jnp.float32),
            jax.ShapeDtypeStruct((batch * E,), jnp.float32),
        ),
        mesh=mesh,
        compiler_params=pltpu.CompilerParams(
            needs_layout_passes=False, disable_bounds_checks=True),
        scratch_types=[
            pltpu.VMEM((DV * N,), jnp.int32),
            pltpu.VMEM((2 * L,), jnp.float32),
            pltpu.VMEM((_GI * N,), jnp.float32),
            pltpu.VMEM((_GI * N,), jnp.float32),
            pltpu.VMEM((_GI * E,), jnp.float32),
            pltpu.VMEM((_GI * E,), jnp.float32),
            pltpu.SemaphoreType.DMA,
            pltpu.SemaphoreType.DMA,
            pltpu.SemaphoreType.DMA,
            pltpu.SemaphoreType.DMA,
            pltpu.SemaphoreType.DMA,
            pltpu.SemaphoreType.DMA,
            pltpu.SemaphoreType.DMA,
        ],
    )
    def k(noise_hbm, tab_hbm, coef_hbm, llr_hbm, msg_hbm,
          tab_v, coef_v, nz0, nz1, mo0, mo1,
          tsem, is0, is1, ol0, ol1, om0, om1):
        nz = (nz0, nz1)
        mo = (mo0, mo1)
        isem = (is0, is1)
        olsem = (ol0, ol1)
        omsem = (om0, om1)
        wid = lax.axis_index("s") * NC + lax.axis_index("c")
        base = wid * rows_w
        tc1 = pltpu.async_copy(tab_hbm, tab_v, tsem)
        tc2 = pltpu.async_copy(coef_hbm, coef_v, tsem)

        def start_in(g):
            b = g % 2
            r0 = base + g * _GI
            return pltpu.async_copy(
                noise_hbm.at[pl.ds(r0 * N, _GI * N)], nz[b], isem[b])

        pending_in = {0: start_in(0)}
        pending_ol = {}
        pending_om = {}
        tc1.wait()
        tc2.wait()
        for g in range(ngroups):
            b = g % 2
            if g + 1 < ngroups:
                if g - 1 in pending_ol:
                    pending_ol.pop(g - 1).wait()
                pending_in[g + 1] = start_in(g + 1)
            pending_in.pop(g).wait()
            if g - 2 in pending_om:
                pending_om.pop(g - 2).wait()
            nzb, mob = nz[b], mo[b]

            @plsc.parallel_loop(0, N // L, unroll=1)
            def vchunk(i):
                o = i * L
                cmu = coef_v[pl.ds(0, L)]
                cs = coef_v[pl.ds(L, L)]
                a0 = tab_v[pl.ds(o, L)]
                a1 = tab_v[pl.ds(N + o, L)]
                a2 = tab_v[pl.ds(2 * N + o, L)]
                ev = jnp.full((L,), E, jnp.int32)
                for r in range(_GI):
                    x = cmu + cs * nzb[pl.ds(r * N + o, L)]
                    nzb[pl.ds(r * N + o, L)] = x
                    plsc.store_scatter(mob, [a0], x)
                    plsc.store_scatter(mob, [a1], x)
                    plsc.store_scatter(mob, [a2], x)
                    if r + 1 < _GI:
                        a0 = a0 + ev
                        a1 = a1 + ev
                        a2 = a2 + ev

            r0 = base + g * _GI
            pending_ol[g] = pltpu.async_copy(
                nzb, llr_hbm.at[pl.ds(r0 * N, _GI * N)], olsem[b])
            pending_om[g] = pltpu.async_copy(
                mob, msg_hbm.at[pl.ds(r0 * E, _GI * E)], omsem[b])
        for d in (pending_ol, pending_om):
            for g in sorted(d):
                d[g].wait()

    return k


def _sc_step_build(batch):
    rows_w = batch // NW
    ngroups = rows_w // _GS
    mesh = plsc.VectorSubcoreMesh(core_axis_name="c", subcore_axis_name="s")

    @functools.partial(
        pl.kernel,
        out_type=(
            jax.ShapeDtypeStruct((batch * N,), jnp.float32),
            jax.ShapeDtypeStruct((batch * E,), jnp.float32),
        ),
        mesh=mesh,
        compiler_params=pltpu.CompilerParams(
            needs_layout_passes=False, disable_bounds_checks=True),
        scratch_types=[
            pltpu.VMEM((DV * N,), jnp.int32),
            pltpu.VMEM((_GS * E,), jnp.float32),
            pltpu.VMEM((_GS * E,), jnp.float32),
            pltpu.VMEM((_GS * N,), jnp.float32),
            pltpu.VMEM((_GS * N,), jnp.float32),
            pltpu.VMEM((_GS * N,), jnp.float32),
            pltpu.VMEM((_GS * E,), jnp.float32),
            pltpu.VMEM((_GS * E,), jnp.float32),
            pltpu.SemaphoreType.DMA,
            pltpu.SemaphoreType.DMA,
            pltpu.SemaphoreType.DMA,
            pltpu.SemaphoreType.DMA,
            pltpu.SemaphoreType.DMA,
            pltpu.SemaphoreType.DMA,
        ],
    )
    def k(msgcn_hbm, llrdec_hbm, tab_hbm, llrout_hbm, msg_hbm,
          tab_v, mc0, mc1, ll0, ll1, lo_v, mo0, mo1,
          tsem, is0, is1, oslo, osm0, osm1):
        mc = (mc0, mc1)
        ll = (ll0, ll1)
        mo = (mo0, mo1)
        isem = (is0, is1)
        osem = (osm0, osm1)
        wid = lax.axis_index("s") * NC + lax.axis_index("c")
        base = wid * rows_w
        tcopy = pltpu.async_copy(tab_hbm, tab_v, tsem)

        def start_in(g):
            b = g % 2
            r0 = base + g * _GS
            c1 = pltpu.async_copy(
                msgcn_hbm.at[pl.ds(r0 * E, _GS * E)], mc[b], isem[b])
            c2 = pltpu.async_copy(
                llrdec_hbm.at[pl.ds(r0 * N, _GS * N)], ll[b], isem[b])
            return (c1, c2)

        pending_in = {0: start_in(0)}
        pending_lo = None
        pending_mo = {}
        tcopy.wait()
        for g in range(ngroups):
            b = g % 2
            if g + 1 < ngroups:
                pending_in[g + 1] = start_in(g + 1)
            for c in pending_in.pop(g):
                c.wait()
            if pending_lo is not None:
                pending_lo.wait()
            if g - 2 in pending_mo:
                pending_mo.pop(g - 2).wait()
            mcb, llb, mob = mc[b], ll[b], mo[b]

            @plsc.parallel_loop(0, N // L, unroll=1)
            def vchunk(i):
                o = i * L
                a0 = tab_v[pl.ds(o, L)]
                a1 = tab_v[pl.ds(N + o, L)]
                a2 = tab_v[pl.ds(2 * N + o, L)]
                ev = jnp.full((L,), E, jnp.int32)
                for r in range(_GS):
                    m0 = plsc.load_gather(mcb, [a0])
                    m1 = plsc.load_gather(mcb, [a1])
                    m2 = plsc.load_gather(mcb, [a2])
                    s = m0 + m1
                    s = s + m2
                    x = llb[pl.ds(r * N + o, L)] + s
                    lo_v[pl.ds(r * N + o, L)] = x
                    plsc.store_scatter(mob, [a0], x - m0)
                    plsc.store_scatter(mob, [a1], x - m1)
                    plsc.store_scatter(mob, [a2], x - m2)
                    if r + 1 < _GS:
                        a0 = a0 + ev
                        a1 = a1 + ev
                        a2 = a2 + ev

            r0 = base + g * _GS
            pending_lo = pltpu.async_copy(
                lo_v, llrout_hbm.at[pl.ds(r0 * N, _GS * N)], oslo)
            pending_mo[g] = pltpu.async_copy(
                mob, msg_hbm.at[pl.ds(r0 * E, _GS * E)], osem[b])
        pending_lo.wait()
        for g in sorted(pending_mo):
            pending_mo[g].wait()

    return k


def kernel(batch_size, ebno_db, edge_weights, llr_noise):
    batch = llr_noise.shape[0]
    ebno_lin = 10.0 ** (ebno_db / 10.0)
    no = 1.0 / (ebno_lin * BITS_PER_SYM * CODERATE)
    sigma2 = 4.0 / no
    mu = sigma2 / 2.0
    s = jnp.sqrt(sigma2)

    w4 = edge_weights.reshape(-1)[jnp.asarray(_PERM_NP)].reshape(1, 3, 8, 128)
    tab = jnp.asarray(_POS3_NP.reshape(-1))

    noise_f = llr_noise.reshape(-1)
    coef = jnp.concatenate([jnp.full((L,), mu, jnp.float32),
                            jnp.full((L,), s, jnp.float32)])

    b2 = batch // 2
    sc_init = _sc_init_build(b2)
    sc_step = _sc_step_build(b2)

    llrdecs = [None, None]
    msgs = [None, None]
    for h in range(2):
        llrdecs[h], msgs[h] = sc_init(
            noise_f[h * b2 * N:(h + 1) * b2 * N], tab, coef)
    llr_outs = [[], []]
    parts = []
    for it in range(NUM_ITER):
        for h in range(2):
            m4 = msgs[h].reshape(b2, 3, 8, 128)
            if it == 0:
                cn4 = _cn_update(m4, w4)
            else:
                cn4, p = _cn_update_loss(
                    m4, w4, llr_outs[h][-1].reshape(b2, 8, 128))
                parts.append(p)
            lo_f, msgs[h] = sc_step(cn4.reshape(-1), llrdecs[h], tab)
            llr_outs[h].append(lo_f)

    partials = _loss_partials(
        [llr_outs[h][-1].reshape(b2, 8, 128) for h in range(2)])
    loss = (jnp.sum(partials) + sum(jnp.sum(p) for p in parts)) \
        / jnp.float32(NUM_ITER * batch * N)
    batch_dep = (jnp.asarray(batch_size) * 0).astype(jnp.float32)
    c = jnp.zeros((batch, N), jnp.float32) + batch_dep
    c_hat = -jnp.concatenate(
        [llr_outs[0][-1], llr_outs[1][-1]]).reshape(batch, N)
    return (c, c_hat, loss)


# speedup vs baseline: 1.0308x; 1.0308x over previous
import functools

import numpy as np
import jax
import jax.numpy as jnp
from jax import lax
from jax.experimental import pallas as pl
from jax.experimental.pallas import tpu as pltpu
from jax.experimental.pallas import tpu_sc as plsc

N = 1024
M = 512
DV = 3
DC = 6
E = N * DV
NUM_ITER = 5
BITS_PER_SYM = 2
CODERATE = 0.5

NC = 2
NS = 16
NW = NC * NS
L = 16


def _build_tables():
    rng = np.random.RandomState(0)
    cn = rng.permutation(np.repeat(np.arange(M), DC))
    order = np.argsort(cn, kind="stable")
    perm = np.empty(E, np.int64)
    for m in range(M):
        for j in range(DC):
            perm[j * M + m] = order[m * DC + j]
    vn_of = perm // DV
    pos_of_orig = np.empty(E, np.int64)
    pos_of_orig[perm] = np.arange(E)
    pos3 = pos_of_orig.reshape(N, DV).T.copy()
    return (perm.astype(np.int32), vn_of.astype(np.int32), pos3.astype(np.int32))


_PERM_NP, _VNOF_NP, _POS3_NP = _build_tables()


def _phi(x):
    x = jnp.clip(x, 1e-7, 20.0)
    return -jnp.log(jnp.tanh(x * 0.5))


def _half_swap(y):
    return jnp.concatenate([y[:, 4:], y[:, :4]], axis=1)


def _lo_both(y):
    return jnp.concatenate([y[:, :4], y[:, :4]], axis=1)


def _hi_both(y):
    return jnp.concatenate([y[:, 4:], y[:, 4:]], axis=1)


def _cn_math(x, out_ref):
    t = jnp.where(x < 0, -1.0, 1.0)
    ph = _phi(jnp.abs(x))
    ph_s = ph[:, 0] + _half_swap(ph[:, 0])
    ph_s = ph_s + _lo_both(ph[:, 1])
    ph_s = ph_s + _hi_both(ph[:, 1])
    ph_s = ph_s + _lo_both(ph[:, 2])
    ph_s = ph_s + _hi_both(ph[:, 2])
    tp = t[:, 0] * t[:, 1] * t[:, 2]
    t_s = tp * _half_swap(tp)
    for d in range(3):
        out_ref[:, d] = (t_s * t[:, d]) * _phi(ph_s - ph[:, d])


def _cn_body(msg_ref, w_ref, out_ref):
    _cn_math(msg_ref[...] * w_ref[...], out_ref)


def _cn_loss_body(msg_ref, w_ref, lo_ref, out_ref, part_ref):
    _cn_math(msg_ref[...] * w_ref[...], out_ref)
    z = -lo_ref[...]
    sp = jnp.maximum(z, 0.0) + jnp.log(1.0 + jnp.exp(-jnp.abs(z)))
    part_ref[...] = jnp.reshape(jnp.sum(sp), (1, 1, 1))


def _cn_update(msg4, w4, bb=128):
    b = msg4.shape[0]
    return pl.pallas_call(
        _cn_body,
        grid=(b // bb,),
        in_specs=[pl.BlockSpec((bb, 3, 8, 128), lambda i: (i, 0, 0, 0)),
                  pl.BlockSpec((1, 3, 8, 128), lambda i: (0, 0, 0, 0))],
        out_specs=pl.BlockSpec((bb, 3, 8, 128), lambda i: (i, 0, 0, 0)),
        out_shape=jax.ShapeDtypeStruct((b, 3, 8, 128), jnp.float32),
    )(msg4, w4)


def _cn_update_loss(msg4, w4, lo3, bb=128):
    b = msg4.shape[0]
    return pl.pallas_call(
        _cn_loss_body,
        grid=(b // bb,),
        in_specs=[pl.BlockSpec((bb, 3, 8, 128), lambda i: (i, 0, 0, 0)),
                  pl.BlockSpec((1, 3, 8, 128), lambda i: (0, 0, 0, 0)),
                  pl.BlockSpec((bb, 8, 128), lambda i: (i, 0, 0))],
        out_specs=(pl.BlockSpec((bb, 3, 8, 128), lambda i: (i, 0, 0, 0)),
                   pl.BlockSpec((1, 1, 1), lambda i: (i, 0, 0))),
        out_shape=(jax.ShapeDtypeStruct((b, 3, 8, 128), jnp.float32),
                   jax.ShapeDtypeStruct((b // bb, 1, 1), jnp.float32)),
    )(msg4, w4, lo3)


def _loss_body(*refs):
    out_ref = refs[-1]
    s = jnp.float32(0.0)
    for r in refs[:-1]:
        x = -r[...]
        sp = jnp.maximum(x, 0.0) + jnp.log(1.0 + jnp.exp(-jnp.abs(x)))
        s = s + jnp.sum(sp)
    out_ref[...] = jnp.reshape(s, (1, 1, 1))


def _loss_partials(llr_outs, bb=256):
    b = llr_outs[0].shape[0]
    g = b // bb
    return pl.pallas_call(
        _loss_body,
        grid=(g,),
        in_specs=[pl.BlockSpec((bb, 8, 128), lambda i: (i, 0, 0))
                  for _ in llr_outs],
        out_specs=pl.BlockSpec((1, 1, 1), lambda i: (i, 0, 0)),
        out_shape=jax.ShapeDtypeStruct((g, 1, 1), jnp.float32),
    )(*llr_outs)


_GI = 8
_GS = 8


def _group_sizes(rows, gmax):
    if rows >= 2 * gmax and rows % gmax == 0:
        half = gmax // 2
        return [half] + [gmax] * ((rows - gmax) // gmax) + [half]
    return [min(gmax, rows - i) for i in range(0, rows, gmax)]


def _sc_init_build(batch):
    rows_w = batch // NW
    sizes = _group_sizes(rows_w, _GI)
    offs = [sum(sizes[:g]) for g in range(len(sizes))]
    ngroups = len(sizes)
    mesh = plsc.VectorSubcoreMesh(core_axis_name="c", subcore_axis_name="s")

    @functools.partial(
        pl.kernel,
        out_type=(
            jax.ShapeDtypeStruct((batch * N,), jnp.float32),
            jax.ShapeDtypeStruct((batch * E,), jnp.float32),
        ),
        mesh=mesh,
        compiler_params=pltpu.CompilerParams(
            needs_layout_passes=False, disable_bounds_checks=True),
        scratch_types=[
            pltpu.VMEM((DV * N,), jnp.int32),
            pltpu.VMEM((2 * L,), jnp.float32),
            pltpu.VMEM((_GI * N,), jnp.float32),
            pltpu.VMEM((_GI * N,), jnp.float32),
            pltpu.VMEM((_GI * E,), jnp.float32),
            pltpu.VMEM((_GI * E,), jnp.float32),
            pltpu.SemaphoreType.DMA,
            pltpu.SemaphoreType.DMA,
            pltpu.SemaphoreType.DMA,
            pltpu.SemaphoreType.DMA,
            pltpu.SemaphoreType.DMA,
            pltpu.SemaphoreType.DMA,
            pltpu.SemaphoreType.DMA,
        ],
    )
    def k(noise_hbm, tab_hbm, coef_hbm, llr_hbm, msg_hbm,
          tab_v, coef_v, nz0, nz1, mo0, mo1,
          tsem, is0, is1, ol0, ol1, om0, om1):
        nz = (nz0, nz1)
        mo = (mo0, mo1)
        isem = (is0, is1)
        olsem = (ol0, ol1)
        omsem = (om0, om1)
        wid = lax.axis_index("s") * NC + lax.axis_index("c")
        base = wid * rows_w
        tc1 = pltpu.async_copy(tab_hbm, tab_v, tsem)
        tc2 = pltpu.async_copy(coef_hbm, coef_v, tsem)

        def start_in(g):
            b = g % 2
            r0 = base + offs[g]
            return pltpu.async_copy(
                noise_hbm.at[pl.ds(r0 * N, sizes[g] * N)],
                nz[b].at[pl.ds(0, sizes[g] * N)], isem[b])

        pending_in = {0: start_in(0)}
        pending_ol = {}
        pending_om = {}
        tc1.wait()
        tc2.wait()
        for g in range(ngroups):
            b = g % 2
            sz = sizes[g]
            if g + 1 < ngroups:
                if g - 1 in pending_ol:
                    pending_ol.pop(g - 1).wait()
                pending_in[g + 1] = start_in(g + 1)
            pending_in.pop(g).wait()
            if g - 2 in pending_om:
                pending_om.pop(g - 2).wait()
            nzb, mob = nz[b], mo[b]

            @plsc.parallel_loop(0, N // L, unroll=1)
            def vchunk(i):
                o = i * L
                cmu = coef_v[pl.ds(0, L)]
                cs = coef_v[pl.ds(L, L)]
                a0 = tab_v[pl.ds(o, L)]
                a1 = tab_v[pl.ds(N + o, L)]
                a2 = tab_v[pl.ds(2 * N + o, L)]
                ev = jnp.full((L,), E, jnp.int32)
                for r in range(sz):
                    x = cmu + cs * nzb[pl.ds(r * N + o, L)]
                    nzb[pl.ds(r * N + o, L)] = x
                    plsc.store_scatter(mob, [a0], x)
                    plsc.store_scatter(mob, [a1], x)
                    plsc.store_scatter(mob, [a2], x)
                    if r + 1 < sz:
                        a0 = a0 + ev
                        a1 = a1 + ev
                        a2 = a2 + ev

            r0 = base + offs[g]
            pending_ol[g] = pltpu.async_copy(
                nzb.at[pl.ds(0, sz * N)],
                llr_hbm.at[pl.ds(r0 * N, sz * N)], olsem[b])
            pending_om[g] = pltpu.async_copy(
                mob.at[pl.ds(0, sz * E)],
                msg_hbm.at[pl.ds(r0 * E, sz * E)], omsem[b])
        for d in (pending_ol, pending_om):
            for g in sorted(d):
                d[g].wait()

    return k


def _sc_step_build(batch):
    rows_w = batch // NW
    sizes = _group_sizes(rows_w, _GS)
    offs = [sum(sizes[:g]) for g in range(len(sizes))]
    ngroups = len(sizes)
    mesh = plsc.VectorSubcoreMesh(core_axis_name="c", subcore_axis_name="s")

    @functools.partial(
        pl.kernel,
        out_type=(
            jax.ShapeDtypeStruct((batch * N,), jnp.float32),
            jax.ShapeDtypeStruct((batch * E,), jnp.float32),
        ),
        mesh=mesh,
        compiler_params=pltpu.CompilerParams(
            needs_layout_passes=False, disable_bounds_checks=True),
        scratch_types=[
            pltpu.VMEM((DV * N,), jnp.int32),
            pltpu.VMEM((_GS * E,), jnp.float32),
            pltpu.VMEM((_GS * E,), jnp.float32),
            pltpu.VMEM((_GS * N,), jnp.float32),
            pltpu.VMEM((_GS * N,), jnp.float32),
            pltpu.VMEM((_GS * N,), jnp.float32),
            pltpu.VMEM((_GS * E,), jnp.float32),
            pltpu.VMEM((_GS * E,), jnp.float32),
            pltpu.SemaphoreType.DMA,
            pltpu.SemaphoreType.DMA,
            pltpu.SemaphoreType.DMA,
            pltpu.SemaphoreType.DMA,
            pltpu.SemaphoreType.DMA,
            pltpu.SemaphoreType.DMA,
        ],
    )
    def k(msgcn_hbm, llrdec_hbm, tab_hbm, llrout_hbm, msg_hbm,
          tab_v, mc0, mc1, ll0, ll1, lo_v, mo0, mo1,
          tsem, is0, is1, oslo, osm0, osm1):
        mc = (mc0, mc1)
        ll = (ll0, ll1)
        mo = (mo0, mo1)
        isem = (is0, is1)
        osem = (osm0, osm1)
        wid = lax.axis_index("s") * NC + lax.axis_index("c")
        base = wid * rows_w
        tcopy = pltpu.async_copy(tab_hbm, tab_v, tsem)

        def start_in(g):
            b = g % 2
            r0 = base + offs[g]
            c1 = pltpu.async_copy(
                msgcn_hbm.at[pl.ds(r0 * E, sizes[g] * E)],
                mc[b].at[pl.ds(0, sizes[g] * E)], isem[b])
            c2 = pltpu.async_copy(
                llrdec_hbm.at[pl.ds(r0 * N, sizes[g] * N)],
                ll[b].at[pl.ds(0, sizes[g] * N)], isem[b])
            return (c1, c2)

        pending_in = {0: start_in(0)}
        pending_lo = None
        pending_mo = {}
        tcopy.wait()
        for g in range(ngroups):
            b = g % 2
            if g + 1 < ngroups:
                pending_in[g + 1] = start_in(g + 1)
            for c in pending_in.pop(g):
                c.wait()
            if pending_lo is not None:
                pending_lo.wait()
            if g - 2 in pending_mo:
                pending_mo.pop(g - 2).wait()
            mcb, llb, mob = mc[b], ll[b], mo[b]
            sz = sizes[g]

            @plsc.parallel_loop(0, N // L, unroll=1)
            def vchunk(i):
                o = i * L
                a0 = tab_v[pl.ds(o, L)]
                a1 = tab_v[pl.ds(N + o, L)]
                a2 = tab_v[pl.ds(2 * N + o, L)]
                ev = jnp.full((L,), E, jnp.int32)
                for r in range(sz):
                    m0 = plsc.load_gather(mcb, [a0])
                    m1 = plsc.load_gather(mcb, [a1])
                    m2 = plsc.load_gather(mcb, [a2])
                    s = m0 + m1
                    s = s + m2
                    x = llb[pl.ds(r * N + o, L)] + s
                    lo_v[pl.ds(r * N + o, L)] = x
                    plsc.store_scatter(mob, [a0], x - m0)
                    plsc.store_scatter(mob, [a1], x - m1)
                    plsc.store_scatter(mob, [a2], x - m2)
                    if r + 1 < sz:
                        a0 = a0 + ev
                        a1 = a1 + ev
                        a2 = a2 + ev

            r0 = base + offs[g]
            pending_lo = pltpu.async_copy(
                lo_v.at[pl.ds(0, sz * N)],
                llrout_hbm.at[pl.ds(r0 * N, sz * N)], oslo)
            pending_mo[g] = pltpu.async_copy(
                mob.at[pl.ds(0, sz * E)],
                msg_hbm.at[pl.ds(r0 * E, sz * E)], osem[b])
        pending_lo.wait()
        for g in sorted(pending_mo):
            pending_mo[g].wait()

    return k


def kernel(batch_size, ebno_db, edge_weights, llr_noise):
    batch = llr_noise.shape[0]
    ebno_lin = 10.0 ** (ebno_db / 10.0)
    no = 1.0 / (ebno_lin * BITS_PER_SYM * CODERATE)
    sigma2 = 4.0 / no
    mu = sigma2 / 2.0
    s = jnp.sqrt(sigma2)

    w4 = edge_weights.reshape(-1)[jnp.asarray(_PERM_NP)].reshape(1, 3, 8, 128)
    tab = jnp.asarray(_POS3_NP.reshape(-1))

    noise_f = llr_noise.reshape(-1)
    coef = jnp.concatenate([jnp.full((L,), mu, jnp.float32),
                            jnp.full((L,), s, jnp.float32)])

    b2 = batch // 2
    sc_init = _sc_init_build(b2)
    sc_step = _sc_step_build(b2)

    llrdecs = [None, None]
    msgs = [None, None]
    for h in range(2):
        llrdecs[h], msgs[h] = sc_init(
            noise_f[h * b2 * N:(h + 1) * b2 * N], tab, coef)
    llr_outs = [[], []]
    parts = []
    for it in range(NUM_ITER):
        for h in range(2):
            m4 = msgs[h].reshape(b2, 3, 8, 128)
            if it == 0:
                cn4 = _cn_update(m4, w4)
            else:
                cn4, p = _cn_update_loss(
                    m4, w4, llr_outs[h][-1].reshape(b2, 8, 128))
                parts.append(p)
            lo_f, msgs[h] = sc_step(cn4.reshape(-1), llrdecs[h], tab)
            llr_outs[h].append(lo_f)

    partials = _loss_partials(
        [llr_outs[h][-1].reshape(b2, 8, 128) for h in range(2)])
    loss = (jnp.sum(partials) + sum(jnp.sum(p) for p in parts)) \
        / jnp.float32(NUM_ITER * batch * N)
    batch_dep = (jnp.asarray(batch_size) * 0).astype(jnp.float32)
    c = jnp.zeros((batch, N), jnp.float32) + batch_dep
    c_hat = -jnp.concatenate(
        [llr_outs[0][-1], llr_outs[1][-1]]).reshape(batch, N)
    return (c, c_hat, loss)


# speedup vs baseline: 1.1238x; 1.0903x over previous
import functools

import numpy as np
import jax
import jax.numpy as jnp
from jax import lax
from jax.experimental import pallas as pl
from jax.experimental.pallas import tpu as pltpu
from jax.experimental.pallas import tpu_sc as plsc

N = 1024
M = 512
DV = 3
DC = 6
E = N * DV
NUM_ITER = 5
BITS_PER_SYM = 2
CODERATE = 0.5

NC = 2
NS = 16
NW = NC * NS
L = 16


def _build_tables():
    rng = np.random.RandomState(0)
    cn = rng.permutation(np.repeat(np.arange(M), DC))
    order = np.argsort(cn, kind="stable")
    perm = np.empty(E, np.int64)
    for m in range(M):
        for j in range(DC):
            perm[j * M + m] = order[m * DC + j]
    vn_of = perm // DV
    pos_of_orig = np.empty(E, np.int64)
    pos_of_orig[perm] = np.arange(E)
    pos3 = pos_of_orig.reshape(N, DV).T.copy()
    return (perm.astype(np.int32), vn_of.astype(np.int32), pos3.astype(np.int32))


_PERM_NP, _VNOF_NP, _POS3_NP = _build_tables()


def _phi(x):
    x = jnp.clip(x, 1e-7, 20.0)
    return -jnp.log(jnp.tanh(x * 0.5))


def _half_swap(y):
    return jnp.concatenate([y[:, 4:], y[:, :4]], axis=1)


def _lo_both(y):
    return jnp.concatenate([y[:, :4], y[:, :4]], axis=1)


def _hi_both(y):
    return jnp.concatenate([y[:, 4:], y[:, 4:]], axis=1)


def _cn_math(x, out_ref):
    t = jnp.where(x < 0, -1.0, 1.0)
    ph = _phi(jnp.abs(x))
    ph_s = ph[:, 0] + _half_swap(ph[:, 0])
    ph_s = ph_s + _lo_both(ph[:, 1])
    ph_s = ph_s + _hi_both(ph[:, 1])
    ph_s = ph_s + _lo_both(ph[:, 2])
    ph_s = ph_s + _hi_both(ph[:, 2])
    tp = t[:, 0] * t[:, 1] * t[:, 2]
    t_s = tp * _half_swap(tp)
    for d in range(3):
        out_ref[:, d] = (t_s * t[:, d]) * _phi(ph_s - ph[:, d])


def _cn_body(msg_ref, w_ref, out_ref):
    _cn_math(msg_ref[...] * w_ref[...], out_ref)


def _cn_loss_body(msg_ref, w_ref, lo_ref, out_ref, part_ref):
    _cn_math(msg_ref[...] * w_ref[...], out_ref)
    z = -lo_ref[...]
    sp = jnp.maximum(z, 0.0) + jnp.log(1.0 + jnp.exp(-jnp.abs(z)))
    part_ref[...] = jnp.reshape(jnp.sum(sp), (1, 1, 1))


def _cn_update(msg4, w4, bb=128):
    b = msg4.shape[0]
    return pl.pallas_call(
        _cn_body,
        grid=(b // bb,),
        in_specs=[pl.BlockSpec((bb, 3, 8, 128), lambda i: (i, 0, 0, 0)),
                  pl.BlockSpec((1, 3, 8, 128), lambda i: (0, 0, 0, 0))],
        out_specs=pl.BlockSpec((bb, 3, 8, 128), lambda i: (i, 0, 0, 0)),
        out_shape=jax.ShapeDtypeStruct((b, 3, 8, 128), jnp.float32),
    )(msg4, w4)


def _cn_update_loss(msg4, w4, lo3, bb=128):
    b = msg4.shape[0]
    return pl.pallas_call(
        _cn_loss_body,
        grid=(b // bb,),
        in_specs=[pl.BlockSpec((bb, 3, 8, 128), lambda i: (i, 0, 0, 0)),
                  pl.BlockSpec((1, 3, 8, 128), lambda i: (0, 0, 0, 0)),
                  pl.BlockSpec((bb, 8, 128), lambda i: (i, 0, 0))],
        out_specs=(pl.BlockSpec((bb, 3, 8, 128), lambda i: (i, 0, 0, 0)),
                   pl.BlockSpec((1, 1, 1), lambda i: (i, 0, 0))),
        out_shape=(jax.ShapeDtypeStruct((b, 3, 8, 128), jnp.float32),
                   jax.ShapeDtypeStruct((b // bb, 1, 1), jnp.float32)),
    )(msg4, w4, lo3)


def _loss_body(*refs):
    out_ref = refs[-1]
    s = jnp.float32(0.0)
    for r in refs[:-1]:
        x = -r[...]
        sp = jnp.maximum(x, 0.0) + jnp.log(1.0 + jnp.exp(-jnp.abs(x)))
        s = s + jnp.sum(sp)
    out_ref[...] = jnp.reshape(s, (1, 1, 1))


def _loss_partials(llr_outs, bb=256):
    b = llr_outs[0].shape[0]
    g = b // bb
    return pl.pallas_call(
        _loss_body,
        grid=(g,),
        in_specs=[pl.BlockSpec((bb, 8, 128), lambda i: (i, 0, 0))
                  for _ in llr_outs],
        out_specs=pl.BlockSpec((1, 1, 1), lambda i: (i, 0, 0)),
        out_shape=jax.ShapeDtypeStruct((g, 1, 1), jnp.float32),
    )(*llr_outs)


_GI = 8
_GS = 8


def _group_sizes(rows, gmax):
    if rows >= 3 * gmax and rows % gmax == 0:
        q = gmax // 4
        return ([q, gmax - q] + [gmax] * ((rows - 3 * gmax) // gmax)
                + [gmax - q, q])
    if rows >= 2 * gmax and rows % gmax == 0:
        half = gmax // 2
        return [half] + [gmax] * ((rows - gmax) // gmax) + [half]
    return [min(gmax, rows - i) for i in range(0, rows, gmax)]


def _sc_init_build(batch):
    rows_w = batch // NW
    sizes = _group_sizes(rows_w, _GI)
    offs = [sum(sizes[:g]) for g in range(len(sizes))]
    ngroups = len(sizes)
    mesh = plsc.VectorSubcoreMesh(core_axis_name="c", subcore_axis_name="s")

    @functools.partial(
        pl.kernel,
        out_type=(
            jax.ShapeDtypeStruct((batch * N,), jnp.float32),
            jax.ShapeDtypeStruct((batch * E,), jnp.float32),
        ),
        mesh=mesh,
        compiler_params=pltpu.CompilerParams(
            needs_layout_passes=False, disable_bounds_checks=True),
        scratch_types=[
            pltpu.VMEM((DV * N,), jnp.int32),
            pltpu.VMEM((2 * L,), jnp.float32),
            pltpu.VMEM((_GI * N,), jnp.float32),
            pltpu.VMEM((_GI * N,), jnp.float32),
            pltpu.VMEM((_GI * E,), jnp.float32),
            pltpu.VMEM((_GI * E,), jnp.float32),
            pltpu.SemaphoreType.DMA,
            pltpu.SemaphoreType.DMA,
            pltpu.SemaphoreType.DMA,
            pltpu.SemaphoreType.DMA,
            pltpu.SemaphoreType.DMA,
            pltpu.SemaphoreType.DMA,
            pltpu.SemaphoreType.DMA,
        ],
    )
    def k(noise_hbm, tab_hbm, coef_hbm, llr_hbm, msg_hbm,
          tab_v, coef_v, nz0, nz1, mo0, mo1,
          tsem, is0, is1, ol0, ol1, om0, om1):
        nz = (nz0, nz1)
        mo = (mo0, mo1)
        isem = (is0, is1)
        olsem = (ol0, ol1)
        omsem = (om0, om1)
        wid = lax.axis_index("s") * NC + lax.axis_index("c")
        base = wid * rows_w
        tc1 = pltpu.async_copy(tab_hbm, tab_v, tsem)
        tc2 = pltpu.async_copy(coef_hbm, coef_v, tsem)

        def start_in(g):
            b = g % 2
            r0 = base + offs[g]
            return pltpu.async_copy(
                noise_hbm.at[pl.ds(r0 * N, sizes[g] * N)],
                nz[b].at[pl.ds(0, sizes[g] * N)], isem[b])

        pending_in = {0: start_in(0)}
        pending_ol = {}
        pending_om = {}
        tc1.wait()
        tc2.wait()
        for g in range(ngroups):
            b = g % 2
            sz = sizes[g]
            if g + 1 < ngroups:
                if g - 1 in pending_ol:
                    pending_ol.pop(g - 1).wait()
                pending_in[g + 1] = start_in(g + 1)
            pending_in.pop(g).wait()
            if g - 2 in pending_om:
                pending_om.pop(g - 2).wait()
            nzb, mob = nz[b], mo[b]

            @plsc.parallel_loop(0, N // L, unroll=1)
            def vchunk(i):
                o = i * L
                cmu = coef_v[pl.ds(0, L)]
                cs = coef_v[pl.ds(L, L)]
                a0 = tab_v[pl.ds(o, L)]
                a1 = tab_v[pl.ds(N + o, L)]
                a2 = tab_v[pl.ds(2 * N + o, L)]
                ev = jnp.full((L,), E, jnp.int32)
                for r in range(sz):
                    x = cmu + cs * nzb[pl.ds(r * N + o, L)]
                    nzb[pl.ds(r * N + o, L)] = x
                    plsc.store_scatter(mob, [a0], x)
                    plsc.store_scatter(mob, [a1], x)
                    plsc.store_scatter(mob, [a2], x)
                    if r + 1 < sz:
                        a0 = a0 + ev
                        a1 = a1 + ev
                        a2 = a2 + ev

            r0 = base + offs[g]
            pending_ol[g] = pltpu.async_copy(
                nzb.at[pl.ds(0, sz * N)],
                llr_hbm.at[pl.ds(r0 * N, sz * N)], olsem[b])
            pending_om[g] = pltpu.async_copy(
                mob.at[pl.ds(0, sz * E)],
                msg_hbm.at[pl.ds(r0 * E, sz * E)], omsem[b])
        for d in (pending_ol, pending_om):
            for g in sorted(d):
                d[g].wait()

    return k


def _sc_step_build(batch):
    rows_w = batch // NW
    sizes = _group_sizes(rows_w, _GS)
    offs = [sum(sizes[:g]) for g in range(len(sizes))]
    ngroups = len(sizes)
    mesh = plsc.VectorSubcoreMesh(core_axis_name="c", subcore_axis_name="s")

    @functools.partial(
        pl.kernel,
        out_type=(
            jax.ShapeDtypeStruct((batch * N,), jnp.float32),
            jax.ShapeDtypeStruct((batch * E,), jnp.float32),
        ),
        mesh=mesh,
        compiler_params=pltpu.CompilerParams(
            needs_layout_passes=False, disable_bounds_checks=True),
        scratch_types=[
            pltpu.VMEM((DV * N,), jnp.int32),
            pltpu.VMEM((_GS * E,), jnp.float32),
            pltpu.VMEM((_GS * E,), jnp.float32),
            pltpu.VMEM((_GS * N,), jnp.float32),
            pltpu.VMEM((_GS * N,), jnp.float32),
            pltpu.VMEM((_GS * N,), jnp.float32),
            pltpu.VMEM((_GS * E,), jnp.float32),
            pltpu.VMEM((_GS * E,), jnp.float32),
            pltpu.SemaphoreType.DMA,
            pltpu.SemaphoreType.DMA,
            pltpu.SemaphoreType.DMA,
            pltpu.SemaphoreType.DMA,
            pltpu.SemaphoreType.DMA,
            pltpu.SemaphoreType.DMA,
        ],
    )
    def k(msgcn_hbm, llrdec_hbm, tab_hbm, llrout_hbm, msg_hbm,
          tab_v, mc0, mc1, ll0, ll1, lo_v, mo0, mo1,
          tsem, is0, is1, oslo, osm0, osm1):
        mc = (mc0, mc1)
        ll = (ll0, ll1)
        mo = (mo0, mo1)
        isem = (is0, is1)
        osem = (osm0, osm1)
        wid = lax.axis_index("s") * NC + lax.axis_index("c")
        base = wid * rows_w
        tcopy = pltpu.async_copy(tab_hbm, tab_v, tsem)

        def start_in(g):
            b = g % 2
            r0 = base + offs[g]
            c1 = pltpu.async_copy(
                msgcn_hbm.at[pl.ds(r0 * E, sizes[g] * E)],
                mc[b].at[pl.ds(0, sizes[g] * E)], isem[b])
            c2 = pltpu.async_copy(
                llrdec_hbm.at[pl.ds(r0 * N, sizes[g] * N)],
                ll[b].at[pl.ds(0, sizes[g] * N)], isem[b])
            return (c1, c2)

        pending_in = {0: start_in(0)}
        pending_lo = None
        pending_mo = {}
        tcopy.wait()
        for g in range(ngroups):
            b = g % 2
            if g + 1 < ngroups:
                pending_in[g + 1] = start_in(g + 1)
            for c in pending_in.pop(g):
                c.wait()
            if pending_lo is not None:
                pending_lo.wait()
            if g - 2 in pending_mo:
                pending_mo.pop(g - 2).wait()
            mcb, llb, mob = mc[b], ll[b], mo[b]
            sz = sizes[g]

            @plsc.parallel_loop(0, N // L, unroll=1)
            def vchunk(i):
                o = i * L
                a0 = tab_v[pl.ds(o, L)]
                a1 = tab_v[pl.ds(N + o, L)]
                a2 = tab_v[pl.ds(2 * N + o, L)]
                ev = jnp.full((L,), E, jnp.int32)
                for r in range(sz):
                    m0 = plsc.load_gather(mcb, [a0])
                    m1 = plsc.load_gather(mcb, [a1])
                    m2 = plsc.load_gather(mcb, [a2])
                    s = m0 + m1
                    s = s + m2
                    x = llb[pl.ds(r * N + o, L)] + s
                    lo_v[pl.ds(r * N + o, L)] = x
                    plsc.store_scatter(mob, [a0], x - m0)
                    plsc.store_scatter(mob, [a1], x - m1)
                    plsc.store_scatter(mob, [a2], x - m2)
                    if r + 1 < sz:
                        a0 = a0 + ev
                        a1 = a1 + ev
                        a2 = a2 + ev

            r0 = base + offs[g]
            pending_lo = pltpu.async_copy(
                lo_v.at[pl.ds(0, sz * N)],
                llrout_hbm.at[pl.ds(r0 * N, sz * N)], oslo)
            pending_mo[g] = pltpu.async_copy(
                mob.at[pl.ds(0, sz * E)],
                msg_hbm.at[pl.ds(r0 * E, sz * E)], osem[b])
        pending_lo.wait()
        for g in sorted(pending_mo):
            pending_mo[g].wait()

    return k


def kernel(batch_size, ebno_db, edge_weights, llr_noise):
    batch = llr_noise.shape[0]
    ebno_lin = 10.0 ** (ebno_db / 10.0)
    no = 1.0 / (ebno_lin * BITS_PER_SYM * CODERATE)
    sigma2 = 4.0 / no
    mu = sigma2 / 2.0
    s = jnp.sqrt(sigma2)

    w4 = edge_weights.reshape(-1)[jnp.asarray(_PERM_NP)].reshape(1, 3, 8, 128)
    tab = jnp.asarray(_POS3_NP.reshape(-1))

    noise_f = llr_noise.reshape(-1)
    coef = jnp.concatenate([jnp.full((L,), mu, jnp.float32),
                            jnp.full((L,), s, jnp.float32)])

    b2 = batch // 2
    sc_init = _sc_init_build(b2)
    sc_step = _sc_step_build(b2)

    llrdecs = [None, None]
    msgs = [None, None]
    for h in range(2):
        llrdecs[h], msgs[h] = sc_init(
            noise_f[h * b2 * N:(h + 1) * b2 * N], tab, coef)
    llr_outs = [[], []]
    parts = []
    for it in range(NUM_ITER):
        for h in range(2):
            m4 = msgs[h].reshape(b2, 3, 8, 128)
            if it == 0:
                cn4 = _cn_update(m4, w4)
            else:
                cn4, p = _cn_update_loss(
                    m4, w4, llr_outs[h][-1].reshape(b2, 8, 128))
                parts.append(p)
            lo_f, msgs[h] = sc_step(cn4.reshape(-1), llrdecs[h], tab)
            llr_outs[h].append(lo_f)

    partials = _loss_partials(
        [llr_outs[h][-1].reshape(b2, 8, 128) for h in range(2)])
    loss = (jnp.sum(partials) + sum(jnp.sum(p) for p in parts)) \
        / jnp.float32(NUM_ITER * batch * N)
    batch_dep = (jnp.asarray(batch_size) * 0).astype(jnp.float32)
    c = jnp.zeros((batch, N), jnp.float32) + batch_dep
    c_hat = -jnp.concatenate(
        [llr_outs[0][-1], llr_outs[1][-1]]).reshape(batch, N)
    return (c, c_hat, loss)
